# baseline (speedup 1.0000x reference)
import functools

import numpy as np

import jax
import jax.numpy as jnp
from jax import lax
from jax.experimental import pallas as pl
from jax.experimental.pallas import tpu as pltpu
from jax.experimental.pallas import tpu_sc as plsc

N = 10000
E = 320000
D_IN = 128
HID = [64, 128, 256]
NCHUNK = [2, 4, 8]
OUT_C = 10
G = 128

NS = 16
EB = 80
BS = 200
NB = N // BS

E_T = E // NS


def _proj_body(hc, nc, h_ref, w_ref, b_ref, *outs):
    cw = hc // nc
    m = jnp.dot(h_ref[...], w_ref[...], preferred_element_type=jnp.float32)
    m = m + b_ref[...]
    for c in range(nc):
        outs[c][...] = m[:, c * cw:(c + 1) * cw].astype(jnp.bfloat16)
        outs[nc + c][...] = jnp.concatenate(
            [m[:, hc + c * cw:hc + (c + 1) * cw],
             m[:, 2 * hc + c * cw:2 * hc + (c + 1) * cw]],
            axis=1).astype(jnp.bfloat16)
    outs[2 * nc][...] = m[:, 3 * hc:4 * hc]


def _make_proj(in_c, hc, nc):
    R = 1000
    cw = hc // nc
    f = jnp.float32
    bf = jnp.bfloat16
    out_specs = ([pl.BlockSpec((R, cw), lambda i: (i, 0)) for _ in range(nc)]
                 + [pl.BlockSpec((R, 2 * cw), lambda i: (i, 0)) for _ in range(nc)]
                 + [pl.BlockSpec((R, hc), lambda i: (i, 0))])
    out_shape = ([jax.ShapeDtypeStruct((N, cw), bf) for _ in range(nc)]
                 + [jax.ShapeDtypeStruct((N, 2 * cw), bf) for _ in range(nc)]
                 + [jax.ShapeDtypeStruct((N, hc), f)])
    return pl.pallas_call(
        functools.partial(_proj_body, hc, nc),
        grid=(N // R,),
        in_specs=[
            pl.BlockSpec((R, in_c), lambda i: (i, 0)),
            pl.BlockSpec((in_c, 4 * hc), lambda i: (0, 0)),
            pl.BlockSpec((1, 4 * hc), lambda i: (0, 0)),
        ],
        out_specs=out_specs,
        out_shape=out_shape,
    )


NBUF = 3
NIDX = 2 * NBUF
NBLK = E_T // EB
MAIN_HI = -(-NBLK // NIDX) * NIDX


def _make_edge(hc, nc, stage_qv=False):
    cw = hc // nc
    f = jnp.float32
    mesh = plsc.VectorSubcoreMesh(core_axis_name="c", subcore_axis_name="s",
                                  num_cores=2, num_subcores=NS)

    def body(*refs):
        kts = refs[0:nc]
        qvs = refs[nc:2 * nc]
        src_hbm, dst_hbm = refs[2 * nc], refs[2 * nc + 1]
        outs = refs[2 * nc + 2:3 * nc + 2]
        it = iter(refs[3 * nc + 2:])
        sis = [next(it) for _ in range(NIDX)]
        dis = [next(it) for _ in range(NIDX)]
        kbs = [next(it) for _ in range(NBUF)]
        qbs = [next(it) for _ in range(NBUF)]
        mbs = [next(it) for _ in range(NBUF)]
        zbuf = next(it)
        agg = next(it)
        qvsp = next(it) if stage_qv else None
        sems_i = [next(it) for _ in range(NIDX)]
        sks = [next(it) for _ in range(NBUF)]
        sqs = [next(it) for _ in range(NBUF)]
        sss = [next(it) for _ in range(NBUF)]

        c = lax.axis_index("c")
        s = lax.axis_index("s")

        @pl.loop(0, BS)
        def _(r):
            for j in range(cw // 16):
                zbuf[r, pl.ds(j * 16, 16)] = jnp.zeros((16,), f)

        def row_blocks(fn):
            for j in range(-(-NB // NS)):
                blk = s + NS * j

                @pl.when(blk < NB)
                def _():
                    fn(pl.ds(blk * BS, BS))

        def idx_fetch(x, isl):
            base = s * E_T + x * EB
            pltpu.async_copy(src_hbm.at[pl.ds(base, EB)], sis[isl], sems_i[isl])
            pltpu.async_copy(dst_hbm.at[pl.ds(base, EB)], dis[isl], sems_i[isl])

        def idx_wait(x, isl):
            base = s * E_T + x * EB
            pltpu.make_async_copy(src_hbm.at[pl.ds(base, EB)], sis[isl],
                                  sems_i[isl]).wait()
            pltpu.make_async_copy(dst_hbm.at[pl.ds(base, EB)], dis[isl],
                                  sems_i[isl]).wait()

        def process(kt, qvt, out):
            row_blocks(lambda sl: pltpu.sync_copy(zbuf, agg.at[sl]))
            if stage_qv:
                row_blocks(lambda sl: pltpu.sync_copy(qvt.at[sl], qvsp.at[sl]))
                qvt = qvsp
            plsc.subcore_barrier()

            def gath(isl, rsl):
                pltpu.async_copy(kt.at[dis[isl]], kbs[rsl], sks[rsl])
                pltpu.async_copy(qvt.at[sis[isl]], qbs[rsl], sqs[rsl])

            def wait_gath(isl, rsl):
                pltpu.make_async_copy(kt.at[dis[isl]], kbs[rsl], sks[rsl]).wait()
                pltpu.make_async_copy(qvt.at[sis[isl]], qbs[rsl], sqs[rsl]).wait()

            def compute(rsl):
                kb, qb, mb = kbs[rsl], qbs[rsl], mbs[rsl]

                @pl.loop(0, EB)
                def _(b):
                    for g in range(cw // 32):
                        gsl = pl.ds(32 * g, 32)
                        ka, kz = plsc.unpack(kb[b, gsl],
                                             format=plsc.PackFormat.INTERLEAVED)
                        qa, qz = plsc.unpack(qb[b, gsl],
                                             format=plsc.PackFormat.INTERLEAVED)
                        va, vz = plsc.unpack(qb[b, pl.ds(cw + 32 * g, 32)],
                                             format=plsc.PackFormat.INTERLEAVED)
                        mb[b, pl.ds(32 * g, 16)] = (
                            jnp.maximum(ka + qa, 0.0) * va)
                        mb[b, pl.ds(32 * g + 16, 16)] = (
                            jnp.maximum(kz + qz, 0.0) * vz)

            def scat(isl, rsl):
                pltpu.async_copy(mbs[rsl], agg.at[dis[isl]], sss[rsl], add=True)

            def wait_scat(isl, rsl):
                pltpu.make_async_copy(mbs[rsl], agg.at[dis[isl]], sss[rsl]).wait()

            def sub(b, bi, first=False):
                wait_gath(bi % NIDX, bi % NBUF)
                compute(bi % NBUF)
                scat(bi % NIDX, bi % NBUF)
                if not first:
                    wait_scat((bi - 1) % NIDX, (bi + 2) % NBUF)

            def pre(b, bi):
                idx_wait(b + 2, (bi + 2) % NIDX)
                gath((bi + 2) % NIDX, (bi + 2) % NBUF)

            for x in range(5):
                idx_fetch(x, x)
            for x in range(2):
                idx_wait(x, x)
                gath(x, x)
            for b in range(NIDX):
                sub(b, b, first=(b == 0))
                pre(b, b)
                idx_fetch(b + 5, (b + 5) % NIDX)

            @pl.loop(NIDX, MAIN_HI, step=NIDX)
            def _(j):
                for i in range(NIDX):
                    b = j + i

                    @pl.when(b < NBLK)
                    def _():
                        sub(b, i)

                    @pl.when(b + 2 < NBLK)
                    def _():
                        pre(b, i)

                    @pl.when(b + 5 < NBLK)
                    def _():
                        idx_fetch(b + 5, (i + 5) % NIDX)

            wait_scat((NBLK - 1) % NIDX, (NBLK - 1) % NBUF)

            plsc.subcore_barrier()
            row_blocks(lambda sl: pltpu.sync_copy(agg.at[sl], out.at[sl]))
            plsc.subcore_barrier()

        half = nc // 2

        @pl.when(c == 0)
        def _():
            for ci in range(half):
                process(kts[ci], qvs[ci], outs[ci])

        @pl.when(c == 1)
        def _():
            for ci in range(half, nc):
                process(kts[ci], qvs[ci], outs[ci])

    return pl.kernel(
        body,
        out_type=[jax.ShapeDtypeStruct((N, cw), f) for _ in range(nc)],
        mesh=mesh,
        compiler_params=pltpu.CompilerParams(use_tc_tiling_on_sc=False,
                                             needs_layout_passes=False),
        scratch_types=(
            [pltpu.VMEM((EB,), jnp.int32) for _ in range(2 * NIDX)]
            + [pltpu.VMEM((EB, cw), jnp.bfloat16) for _ in range(NBUF)]
            + [pltpu.VMEM((EB, 2 * cw), jnp.bfloat16) for _ in range(NBUF)]
            + [pltpu.VMEM((EB, cw), f) for _ in range(NBUF)]
            + [pltpu.VMEM((BS, cw), f),
               pltpu.VMEM_SHARED((N, cw), f)]
            + ([pltpu.VMEM_SHARED((N, 2 * cw), jnp.bfloat16)]
               if stage_qv else [])
            + [pltpu.SemaphoreType.DMA for _ in range(NIDX + 3 * NBUF)]
        ),
    )


def _update_body(nc, *refs):
    aggs = refs[0:nc]
    skip_ref, wl_ref, bl_ref, out_ref = refs[nc:]
    a = jnp.concatenate([r[...] for r in aggs], axis=1) + skip_ref[...]
    a = jnp.where(a >= 0.0, a, 0.01 * a)
    out_ref[...] = jnp.dot(a, wl_ref[...], preferred_element_type=jnp.float32) + bl_ref[...]


def _make_update(hc, nc):
    R = 1000
    cw = hc // nc
    f = jnp.float32
    return pl.pallas_call(
        functools.partial(_update_body, nc),
        grid=(N // R,),
        in_specs=(
            [pl.BlockSpec((R, cw), lambda i: (i, 0)) for _ in range(nc)]
            + [
                pl.BlockSpec((R, hc), lambda i: (i, 0)),
                pl.BlockSpec((hc, hc), lambda i: (0, 0)),
                pl.BlockSpec((1, hc), lambda i: (0, 0)),
            ]),
        out_specs=pl.BlockSpec((R, hc), lambda i: (i, 0)),
        out_shape=jax.ShapeDtypeStruct((N, hc), f),
    )


_BN_INV = 0.9999950000374997


def _pool_body(nblk, h_ref, b_ref, w0, b0, w1, b1, w2, b2, w3, b3, w4, b4,
               out_ref, s_scr, c_scr):
    i = pl.program_id(0)
    R = h_ref.shape[0]

    @pl.when(i == 0)
    def _():
        s_scr[...] = jnp.zeros_like(s_scr)
        c_scr[...] = jnp.zeros_like(c_scr)

    ids = jax.lax.broadcasted_iota(jnp.int32, (R, G), 1)
    onehot = (b_ref[...] == ids).astype(jnp.float32)
    dn = (((0,), (0,)), ((), ()))
    s_scr[...] += lax.dot_general(onehot, h_ref[...], dn,
                                  preferred_element_type=jnp.float32)
    c_scr[...] += lax.dot_general(onehot, jnp.ones((R, 8), jnp.float32), dn,
                                  preferred_element_type=jnp.float32)

    @pl.when(i == nblk - 1)
    def _():
        g = s_scr[...] / jnp.maximum(c_scr[:, 0:1], 1.0)
        for w, b in ((w0, b0), (w1, b1), (w2, b2), (w3, b3)):
            g = jnp.dot(g, w[...], preferred_element_type=jnp.float32) + b[...]
            g = jnp.maximum(g * _BN_INV, 0.0)
        out_ref[...] = jnp.dot(g, w4[...], preferred_element_type=jnp.float32) + b4[...]


def _make_pool(hc, dims):
    R = 1000
    nblk = N // R
    f = jnp.float32
    in_specs = [
        pl.BlockSpec((R, hc), lambda i: (i, 0)),
        pl.BlockSpec((R, 1), lambda i: (i, 0)),
    ]
    for k in range(5):
        in_specs.append(pl.BlockSpec((dims[k], dims[k + 1]), lambda i: (0, 0)))
        in_specs.append(pl.BlockSpec((1, dims[k + 1]), lambda i: (0, 0)))
    return pl.pallas_call(
        functools.partial(_pool_body, nblk),
        grid=(nblk,),
        in_specs=in_specs,
        out_specs=pl.BlockSpec((G, OUT_C), lambda i: (0, 0)),
        out_shape=jax.ShapeDtypeStruct((G, OUT_C), f),
        scratch_shapes=[pltpu.VMEM((G, hc), f), pltpu.VMEM((G, 8), f)],
    )


def kernel(x, params, edge_index, batch):
    src = edge_index[0]
    dst = edge_index[1]
    h = x
    in_c = D_IN
    for li, hc in enumerate(HID):
        nc = NCHUNK[li]
        p = lambda nm: params['l%d_%s' % (li, nm)]
        perm = np.arange(hc).reshape(-1, 2, 16).transpose(0, 2, 1).reshape(-1)
        wcat = jnp.concatenate([p('Wk')[:, perm], p('Wq')[:, perm],
                                p('Wv')[:, perm], p('Ws')], axis=1)
        bcat = jnp.concatenate([p('bk')[perm], p('bq')[perm],
                                p('bv')[perm], p('bias')])[None, :]
        proj = _make_proj(in_c, hc, nc)(h, wcat, bcat)
        aggs = _make_edge(hc, nc, stage_qv=(li >= 1))(*proj[:2 * nc], src, dst)
        h = _make_update(hc, nc)(*aggs, proj[2 * nc], p('Wl'), p('bl')[None, :])
        in_c = hc

    dims = [HID[-1], 64, 64, 64, 64, OUT_C]
    args = [h, batch[:, None]]
    for k in range(5):
        args.append(params['m_W%d' % k])
        args.append(params['m_b%d' % k][None, :])
    return _make_pool(HID[-1], dims)(*args)


# speedup vs baseline: 1.0789x; 1.0789x over previous
import functools

import numpy as np

import jax
import jax.numpy as jnp
from jax import lax
from jax.experimental import pallas as pl
from jax.experimental.pallas import tpu as pltpu
from jax.experimental.pallas import tpu_sc as plsc

N = 10000
E = 320000
D_IN = 128
HID = [64, 128, 256]
NCHUNK = [2, 4, 4]
OUT_C = 10
G = 128

NS = 16
EB = 80
BS = 200
NB = N // BS

E_T = E // NS


def _proj_body(hc, nc, h_ref, w_ref, b_ref, *outs):
    cw = hc // nc
    m = jnp.dot(h_ref[...], w_ref[...], preferred_element_type=jnp.float32)
    m = m + b_ref[...]
    for c in range(nc):
        outs[c][...] = m[:, c * cw:(c + 1) * cw].astype(jnp.bfloat16)
        outs[nc + c][...] = jnp.concatenate(
            [m[:, hc + c * cw:hc + (c + 1) * cw],
             m[:, 2 * hc + c * cw:2 * hc + (c + 1) * cw]],
            axis=1).astype(jnp.bfloat16)
    outs[2 * nc][...] = m[:, 3 * hc:4 * hc]


def _make_proj(in_c, hc, nc):
    R = 1000
    cw = hc // nc
    f = jnp.float32
    bf = jnp.bfloat16
    out_specs = ([pl.BlockSpec((R, cw), lambda i: (i, 0)) for _ in range(nc)]
                 + [pl.BlockSpec((R, 2 * cw), lambda i: (i, 0)) for _ in range(nc)]
                 + [pl.BlockSpec((R, hc), lambda i: (i, 0))])
    out_shape = ([jax.ShapeDtypeStruct((N, cw), bf) for _ in range(nc)]
                 + [jax.ShapeDtypeStruct((N, 2 * cw), bf) for _ in range(nc)]
                 + [jax.ShapeDtypeStruct((N, hc), f)])
    return pl.pallas_call(
        functools.partial(_proj_body, hc, nc),
        grid=(N // R,),
        in_specs=[
            pl.BlockSpec((R, in_c), lambda i: (i, 0)),
            pl.BlockSpec((in_c, 4 * hc), lambda i: (0, 0)),
            pl.BlockSpec((1, 4 * hc), lambda i: (0, 0)),
        ],
        out_specs=out_specs,
        out_shape=out_shape,
    )


NBUF = 3
NIDX = 2 * NBUF
NBLK = E_T // EB
MAIN_HI = -(-NBLK // NIDX) * NIDX


def _make_edge(hc, nc, stage_qv=False):
    cw = hc // nc
    f = jnp.float32
    mesh = plsc.VectorSubcoreMesh(core_axis_name="c", subcore_axis_name="s",
                                  num_cores=2, num_subcores=NS)

    def body(*refs):
        kts = refs[0:nc]
        qvs = refs[nc:2 * nc]
        src_hbm, dst_hbm = refs[2 * nc], refs[2 * nc + 1]
        outs = refs[2 * nc + 2:3 * nc + 2]
        it = iter(refs[3 * nc + 2:])
        sis = [next(it) for _ in range(NIDX)]
        dis = [next(it) for _ in range(NIDX)]
        kbs = [next(it) for _ in range(NBUF)]
        qbs = [next(it) for _ in range(NBUF)]
        mbs = [next(it) for _ in range(NBUF)]
        zbuf = next(it)
        agg = next(it)
        qvsp = next(it) if stage_qv else None
        sems_i = [next(it) for _ in range(NIDX)]
        sks = [next(it) for _ in range(NBUF)]
        sqs = [next(it) for _ in range(NBUF)]
        sss = [next(it) for _ in range(NBUF)]

        c = lax.axis_index("c")
        s = lax.axis_index("s")

        @pl.loop(0, BS)
        def _(r):
            for j in range(cw // 16):
                zbuf[r, pl.ds(j * 16, 16)] = jnp.zeros((16,), f)

        def row_blocks(fn):
            for j in range(-(-NB // NS)):
                blk = s + NS * j

                @pl.when(blk < NB)
                def _():
                    fn(pl.ds(blk * BS, BS))

        def idx_fetch(x, isl):
            base = s * E_T + x * EB
            pltpu.async_copy(src_hbm.at[pl.ds(base, EB)], sis[isl], sems_i[isl])
            pltpu.async_copy(dst_hbm.at[pl.ds(base, EB)], dis[isl], sems_i[isl])

        def idx_wait(x, isl):
            base = s * E_T + x * EB
            pltpu.make_async_copy(src_hbm.at[pl.ds(base, EB)], sis[isl],
                                  sems_i[isl]).wait()
            pltpu.make_async_copy(dst_hbm.at[pl.ds(base, EB)], dis[isl],
                                  sems_i[isl]).wait()

        def process(kt, qvt, out):
            row_blocks(lambda sl: pltpu.sync_copy(zbuf, agg.at[sl]))
            if stage_qv:
                row_blocks(lambda sl: pltpu.sync_copy(qvt.at[sl], qvsp.at[sl]))
                qvt = qvsp
            plsc.subcore_barrier()

            def gath(isl, rsl):
                pltpu.async_copy(kt.at[dis[isl]], kbs[rsl], sks[rsl])
                pltpu.async_copy(qvt.at[sis[isl]], qbs[rsl], sqs[rsl])

            def wait_gath(isl, rsl):
                pltpu.make_async_copy(kt.at[dis[isl]], kbs[rsl], sks[rsl]).wait()
                pltpu.make_async_copy(qvt.at[sis[isl]], qbs[rsl], sqs[rsl]).wait()

            def compute(rsl):
                kb, qb, mb = kbs[rsl], qbs[rsl], mbs[rsl]

                @pl.loop(0, EB)
                def _(b):
                    for g in range(cw // 32):
                        gsl = pl.ds(32 * g, 32)
                        ka, kz = plsc.unpack(kb[b, gsl],
                                             format=plsc.PackFormat.INTERLEAVED)
                        qa, qz = plsc.unpack(qb[b, gsl],
                                             format=plsc.PackFormat.INTERLEAVED)
                        va, vz = plsc.unpack(qb[b, pl.ds(cw + 32 * g, 32)],
                                             format=plsc.PackFormat.INTERLEAVED)
                        mb[b, pl.ds(32 * g, 16)] = (
                            jnp.maximum(ka + qa, 0.0) * va)
                        mb[b, pl.ds(32 * g + 16, 16)] = (
                            jnp.maximum(kz + qz, 0.0) * vz)

            def scat(isl, rsl):
                pltpu.async_copy(mbs[rsl], agg.at[dis[isl]], sss[rsl], add=True)

            def wait_scat(isl, rsl):
                pltpu.make_async_copy(mbs[rsl], agg.at[dis[isl]], sss[rsl]).wait()

            def sub(b, bi, first=False):
                wait_gath(bi % NIDX, bi % NBUF)
                compute(bi % NBUF)
                scat(bi % NIDX, bi % NBUF)
                if not first:
                    wait_scat((bi - 1) % NIDX, (bi + 2) % NBUF)

            def pre(b, bi):
                idx_wait(b + 2, (bi + 2) % NIDX)
                gath((bi + 2) % NIDX, (bi + 2) % NBUF)

            for x in range(5):
                idx_fetch(x, x)
            for x in range(2):
                idx_wait(x, x)
                gath(x, x)
            for b in range(NIDX):
                sub(b, b, first=(b == 0))
                pre(b, b)
                idx_fetch(b + 5, (b + 5) % NIDX)

            @pl.loop(NIDX, MAIN_HI, step=NIDX)
            def _(j):
                for i in range(NIDX):
                    b = j + i

                    @pl.when(b < NBLK)
                    def _():
                        sub(b, i)

                    @pl.when(b + 2 < NBLK)
                    def _():
                        pre(b, i)

                    @pl.when(b + 5 < NBLK)
                    def _():
                        idx_fetch(b + 5, (i + 5) % NIDX)

            wait_scat((NBLK - 1) % NIDX, (NBLK - 1) % NBUF)

            plsc.subcore_barrier()
            row_blocks(lambda sl: pltpu.sync_copy(agg.at[sl], out.at[sl]))
            plsc.subcore_barrier()

        half = nc // 2

        @pl.when(c == 0)
        def _():
            for ci in range(half):
                process(kts[ci], qvs[ci], outs[ci])

        @pl.when(c == 1)
        def _():
            for ci in range(half, nc):
                process(kts[ci], qvs[ci], outs[ci])

    return pl.kernel(
        body,
        out_type=[jax.ShapeDtypeStruct((N, cw), f) for _ in range(nc)],
        mesh=mesh,
        compiler_params=pltpu.CompilerParams(use_tc_tiling_on_sc=False,
                                             needs_layout_passes=False),
        scratch_types=(
            [pltpu.VMEM((EB,), jnp.int32) for _ in range(2 * NIDX)]
            + [pltpu.VMEM((EB, cw), jnp.bfloat16) for _ in range(NBUF)]
            + [pltpu.VMEM((EB, 2 * cw), jnp.bfloat16) for _ in range(NBUF)]
            + [pltpu.VMEM((EB, cw), f) for _ in range(NBUF)]
            + [pltpu.VMEM((BS, cw), f),
               pltpu.VMEM_SHARED((N, cw), f)]
            + ([pltpu.VMEM_SHARED((N, 2 * cw), jnp.bfloat16)]
               if stage_qv else [])
            + [pltpu.SemaphoreType.DMA for _ in range(NIDX + 3 * NBUF)]
        ),
    )


def _update_body(nc, *refs):
    aggs = refs[0:nc]
    skip_ref, wl_ref, bl_ref, out_ref = refs[nc:]
    a = jnp.concatenate([r[...] for r in aggs], axis=1) + skip_ref[...]
    a = jnp.where(a >= 0.0, a, 0.01 * a)
    out_ref[...] = jnp.dot(a, wl_ref[...], preferred_element_type=jnp.float32) + bl_ref[...]


def _make_update(hc, nc):
    R = 1000
    cw = hc // nc
    f = jnp.float32
    return pl.pallas_call(
        functools.partial(_update_body, nc),
        grid=(N // R,),
        in_specs=(
            [pl.BlockSpec((R, cw), lambda i: (i, 0)) for _ in range(nc)]
            + [
                pl.BlockSpec((R, hc), lambda i: (i, 0)),
                pl.BlockSpec((hc, hc), lambda i: (0, 0)),
                pl.BlockSpec((1, hc), lambda i: (0, 0)),
            ]),
        out_specs=pl.BlockSpec((R, hc), lambda i: (i, 0)),
        out_shape=jax.ShapeDtypeStruct((N, hc), f),
    )


_BN_INV = 0.9999950000374997


def _pool_body(nblk, h_ref, b_ref, w0, b0, w1, b1, w2, b2, w3, b3, w4, b4,
               out_ref, s_scr, c_scr):
    i = pl.program_id(0)
    R = h_ref.shape[0]

    @pl.when(i == 0)
    def _():
        s_scr[...] = jnp.zeros_like(s_scr)
        c_scr[...] = jnp.zeros_like(c_scr)

    ids = jax.lax.broadcasted_iota(jnp.int32, (R, G), 1)
    onehot = (b_ref[...] == ids).astype(jnp.float32)
    dn = (((0,), (0,)), ((), ()))
    s_scr[...] += lax.dot_general(onehot, h_ref[...], dn,
                                  preferred_element_type=jnp.float32)
    c_scr[...] += lax.dot_general(onehot, jnp.ones((R, 8), jnp.float32), dn,
                                  preferred_element_type=jnp.float32)

    @pl.when(i == nblk - 1)
    def _():
        g = s_scr[...] / jnp.maximum(c_scr[:, 0:1], 1.0)
        for w, b in ((w0, b0), (w1, b1), (w2, b2), (w3, b3)):
            g = jnp.dot(g, w[...], preferred_element_type=jnp.float32) + b[...]
            g = jnp.maximum(g * _BN_INV, 0.0)
        out_ref[...] = jnp.dot(g, w4[...], preferred_element_type=jnp.float32) + b4[...]


def _make_pool(hc, dims):
    R = 1000
    nblk = N // R
    f = jnp.float32
    in_specs = [
        pl.BlockSpec((R, hc), lambda i: (i, 0)),
        pl.BlockSpec((R, 1), lambda i: (i, 0)),
    ]
    for k in range(5):
        in_specs.append(pl.BlockSpec((dims[k], dims[k + 1]), lambda i: (0, 0)))
        in_specs.append(pl.BlockSpec((1, dims[k + 1]), lambda i: (0, 0)))
    return pl.pallas_call(
        functools.partial(_pool_body, nblk),
        grid=(nblk,),
        in_specs=in_specs,
        out_specs=pl.BlockSpec((G, OUT_C), lambda i: (0, 0)),
        out_shape=jax.ShapeDtypeStruct((G, OUT_C), f),
        scratch_shapes=[pltpu.VMEM((G, hc), f), pltpu.VMEM((G, 8), f)],
    )


def kernel(x, params, edge_index, batch):
    src = edge_index[0]
    dst = edge_index[1]
    h = x
    in_c = D_IN
    for li, hc in enumerate(HID):
        nc = NCHUNK[li]
        p = lambda nm: params['l%d_%s' % (li, nm)]
        perm = np.arange(hc).reshape(-1, 2, 16).transpose(0, 2, 1).reshape(-1)
        wcat = jnp.concatenate([p('Wk')[:, perm], p('Wq')[:, perm],
                                p('Wv')[:, perm], p('Ws')], axis=1)
        bcat = jnp.concatenate([p('bk')[perm], p('bq')[perm],
                                p('bv')[perm], p('bias')])[None, :]
        proj = _make_proj(in_c, hc, nc)(h, wcat, bcat)
        aggs = _make_edge(hc, nc)(*proj[:2 * nc], src, dst)
        h = _make_update(hc, nc)(*aggs, proj[2 * nc], p('Wl'), p('bl')[None, :])
        in_c = hc

    dims = [HID[-1], 64, 64, 64, 64, OUT_C]
    args = [h, batch[:, None]]
    for k in range(5):
        args.append(params['m_W%d' % k])
        args.append(params['m_b%d' % k][None, :])
    return _make_pool(HID[-1], dims)(*args)
